# baseline (speedup 1.0000x reference)
import functools

import jax
import jax.numpy as jnp
from jax import lax
from jax.experimental import pallas as pl
from jax.experimental.pallas import tpu as pltpu
from jax.experimental.pallas import tpu_sc as plsc

_N_MODELS = 1000
_CAP = 64
_EMB = 512
_HALF = 256
_BATCH = 16384

_info = plsc.get_sparse_core_info()
_NC = _info.num_cores
_NS = _info.num_subcores
_NW = _NC * _NS
_CH = 32
_RPW = _BATCH // _NW
_NCHUNK = _RPW // _CH
_NBUF = 7


def _table_body(idt, cap, w1, b1, w2, b2, fw1, fb1, fw2, fb2, out):
    h = jnp.maximum(
        jnp.dot(cap[...], w1[...], preferred_element_type=jnp.float32) + b1[...],
        0.0,
    )
    cap_emb = jnp.dot(h, w2[...], preferred_element_type=jnp.float32) + b2[...]
    me = jnp.concatenate([idt[...], cap_emb], axis=1)
    h2 = jnp.maximum(
        jnp.dot(me, fw1[...], preferred_element_type=jnp.float32) + fb1[...],
        0.0,
    )
    out[...] = jnp.dot(h2, fw2[...], preferred_element_type=jnp.float32) + fb2[...]


def _build_table(id_table, cap_vecs, w1, b1, w2, b2, fw1, fb1, fw2, fb2):
    return pl.pallas_call(
        _table_body,
        out_shape=jax.ShapeDtypeStruct((_N_MODELS, _EMB), jnp.float32),
    )(id_table, cap_vecs, w1, b1, w2, b2, fw1, fb1, fw2, fb2)


@functools.partial(
    pl.kernel,
    out_type=jax.ShapeDtypeStruct((_BATCH, _EMB), jnp.float32),
    mesh=plsc.VectorSubcoreMesh(core_axis_name="c", subcore_axis_name="s"),
    scratch_types=(
        [pltpu.VMEM((_RPW,), jnp.int32)]
        + [pltpu.VMEM((_CH, _EMB), jnp.float32) for _ in range(_NBUF)]
        + [pltpu.SemaphoreType.DMA for _ in range(2 * _NBUF + 2)]
    ),
)
def _sc_gather(table_hbm, idx_hbm, out_hbm, idx_v, *bufs_and_sems):
    bufs = bufs_and_sems[:_NBUF]
    gsems = bufs_and_sems[_NBUF:2 * _NBUF]
    osems = bufs_and_sems[2 * _NBUF:3 * _NBUF]
    isems = bufs_and_sems[3 * _NBUF:]
    wid = lax.axis_index("s") * _NC + lax.axis_index("c")
    base = wid * _RPW
    half = _RPW // 2
    i1 = pltpu.async_copy(idx_hbm.at[pl.ds(base, half)],
                          idx_v.at[pl.ds(0, half)], isems[0])
    i2 = pltpu.async_copy(idx_hbm.at[pl.ds(base + half, half)],
                          idx_v.at[pl.ds(half, half)], isems[1])
    i1.wait()

    def _gather(c):
        return pltpu.async_copy(
            table_hbm.at[idx_v.at[pl.ds(c * _CH, _CH)]], bufs[c % _NBUF],
            gsems[c % _NBUF])

    g = [None] * _NCHUNK
    o = [None] * _NCHUNK
    for c in range(min(_NBUF, _NCHUNK)):
        g[c] = _gather(c)
    for c in range(_NCHUNK):
        b = c % _NBUF
        g[c].wait()
        o[c] = pltpu.async_copy(
            bufs[b], out_hbm.at[pl.ds(base + c * _CH, _CH)], osems[b])
        n = c + 5
        if _NBUF <= n < _NCHUNK:
            if n == _NCHUNK // 2:
                i2.wait()
            o[n - _NBUF].wait()
            g[n] = _gather(n)
    for c in range(max(_NCHUNK - _NBUF, 0), _NCHUNK):
        o[c].wait()


def kernel(model_indices, id_table, capability_vectors, cp_w1, cp_b1, cp_w2,
           cp_b2, fp_w1, fp_b1, fp_w2, fp_b2):
    table = _build_table(
        id_table, capability_vectors,
        cp_w1, cp_b1.reshape(1, _HALF), cp_w2, cp_b2.reshape(1, _HALF),
        fp_w1, fp_b1.reshape(1, _EMB), fp_w2, fp_b2.reshape(1, _EMB),
    )
    idx = model_indices
    if idx.dtype != jnp.int32:
        idx = idx.astype(jnp.int32)
    return _sc_gather(table, idx)


# speedup vs baseline: 1.0338x; 1.0338x over previous
import functools

import jax
import jax.numpy as jnp
from jax import lax
from jax.experimental import pallas as pl
from jax.experimental.pallas import tpu as pltpu
from jax.experimental.pallas import tpu_sc as plsc

_N_MODELS = 1000
_CAP = 64
_EMB = 512
_HALF = 256
_BATCH = 16384

_info = plsc.get_sparse_core_info()
_NC = _info.num_cores
_NS = _info.num_subcores
_NW = _NC * _NS
_CH = 32
_RPW = _BATCH // _NW
_NCHUNK = _RPW // _CH
_NBUF = 7


def _table_body(idt, cap_t, w1, b1, w2, b2, fw1, fb1, fw2, fb2, out):
    h = jnp.maximum(
        lax.dot_general(cap_t[...], w1[...], (((0,), (0,)), ((), ())),
                        preferred_element_type=jnp.float32) + b1[...],
        0.0,
    )
    cap_emb = jnp.dot(h, w2[...], preferred_element_type=jnp.float32) + b2[...]
    me = jnp.concatenate([idt[...], cap_emb], axis=1)
    h2 = jnp.maximum(
        jnp.dot(me, fw1[...], preferred_element_type=jnp.float32) + fb1[...],
        0.0,
    )
    out[...] = jnp.dot(h2, fw2[...], preferred_element_type=jnp.float32) + fb2[...]


def _build_table(id_table, cap_vecs, w1, b1, w2, b2, fw1, fb1, fw2, fb2):
    return pl.pallas_call(
        _table_body,
        out_shape=jax.ShapeDtypeStruct((_N_MODELS, _EMB), jnp.float32),
    )(id_table, cap_vecs, w1, b1, w2, b2, fw1, fb1, fw2, fb2)


@functools.partial(
    pl.kernel,
    out_type=jax.ShapeDtypeStruct((_BATCH, _EMB), jnp.float32),
    mesh=plsc.VectorSubcoreMesh(core_axis_name="c", subcore_axis_name="s"),
    scratch_types=(
        [pltpu.VMEM((_RPW,), jnp.int32)]
        + [pltpu.VMEM((_CH, _EMB), jnp.float32) for _ in range(_NBUF)]
        + [pltpu.SemaphoreType.DMA for _ in range(2 * _NBUF + 2)]
    ),
)
def _sc_gather(table_hbm, idx_hbm, out_hbm, idx_v, *bufs_and_sems):
    bufs = bufs_and_sems[:_NBUF]
    gsems = bufs_and_sems[_NBUF:2 * _NBUF]
    osems = bufs_and_sems[2 * _NBUF:3 * _NBUF]
    isems = bufs_and_sems[3 * _NBUF:]
    wid = lax.axis_index("s") * _NC + lax.axis_index("c")
    base = wid * _RPW
    half = _RPW // 2
    i1 = pltpu.async_copy(idx_hbm.at[pl.ds(base, half)],
                          idx_v.at[pl.ds(0, half)], isems[0])
    i2 = pltpu.async_copy(idx_hbm.at[pl.ds(base + half, half)],
                          idx_v.at[pl.ds(half, half)], isems[1])
    i1.wait()

    def _gather(c):
        return pltpu.async_copy(
            table_hbm.at[idx_v.at[pl.ds(c * _CH, _CH)]], bufs[c % _NBUF],
            gsems[c % _NBUF])

    g = [None] * _NCHUNK
    o = [None] * _NCHUNK
    for c in range(min(_NBUF, _NCHUNK)):
        g[c] = _gather(c)
    for c in range(_NCHUNK):
        b = c % _NBUF
        g[c].wait()
        o[c] = pltpu.async_copy(
            bufs[b], out_hbm.at[pl.ds(base + c * _CH, _CH)], osems[b])
        n = c + 5
        if _NBUF <= n < _NCHUNK:
            if n == _NCHUNK // 2:
                i2.wait()
            o[n - _NBUF].wait()
            g[n] = _gather(n)
    for c in range(max(_NCHUNK - _NBUF, 0), _NCHUNK):
        o[c].wait()


def kernel(model_indices, id_table, capability_vectors, cp_w1, cp_b1, cp_w2,
           cp_b2, fp_w1, fp_b1, fp_w2, fp_b2):
    table = _build_table(
        id_table, capability_vectors.T,
        cp_w1, cp_b1.reshape(1, _HALF), cp_w2, cp_b2.reshape(1, _HALF),
        fp_w1, fp_b1.reshape(1, _EMB), fp_w2, fp_b2.reshape(1, _EMB),
    )
    idx = model_indices
    if idx.dtype != jnp.int32:
        idx = idx.astype(jnp.int32)
    return _sc_gather(table, idx)
